# baseline (speedup 1.0000x reference)
import functools

import jax
import jax.numpy as jnp
from jax import lax
from jax.experimental import pallas as pl
from jax.experimental.pallas import tpu as pltpu
from jax.experimental.pallas import tpu_sc as plsc

N = 10000
E = 320000
D = 128
H = 64

NT = 16
NP = 10240
ROWS_T = NP // NT
EPR = 2560
EPAD = EPR * 128
R = EPR // NT

BR = 1024
NPB = NP // BR

_mesh = plsc.VectorSubcoreMesh(core_axis_name="c", subcore_axis_name="s")


@functools.partial(
    pl.kernel,
    out_type=jax.ShapeDtypeStruct((2, NP, 8), jnp.float32),
    mesh=_mesh,
    compiler_params=pltpu.CompilerParams(use_tc_tiling_on_sc=False),
    scratch_types=[
        pltpu.VMEM((R, 128), jnp.int32),
        pltpu.VMEM((128, 8), jnp.float32),
        pltpu.VMEM_SHARED((NP, 8), jnp.float32),
        pltpu.SemaphoreType.DMA,
    ],
)
def _deg_kernel(dsts, ones8, zeros8, out, idx_v, ones_v, acc_sh, sem):
    c = lax.axis_index("c")
    s = lax.axis_index("s")
    rows0 = s * ROWS_T
    pltpu.sync_copy(dsts.at[c, pl.ds(s * R, R)], idx_v)
    pltpu.sync_copy(ones8, ones_v)
    pltpu.sync_copy(zeros8.at[pl.ds(rows0, ROWS_T)], acc_sh.at[pl.ds(rows0, ROWS_T)])
    plsc.subcore_barrier()

    def fire(j, carry):
        pltpu.async_copy(ones_v, acc_sh.at[idx_v.at[j]], sem, add=True)
        return carry

    lax.fori_loop(0, R, fire, 0)

    def drain(j, carry):
        pltpu.make_async_copy(ones_v, acc_sh.at[idx_v.at[j]], sem).wait()
        return carry

    lax.fori_loop(0, R, drain, 0)
    plsc.subcore_barrier()
    pltpu.sync_copy(acc_sh.at[pl.ds(rows0, ROWS_T)], out.at[c, pl.ds(rows0, ROWS_T)])


@functools.partial(
    pl.kernel,
    out_type=jax.ShapeDtypeStruct((2, NP, H), jnp.float32),
    mesh=_mesh,
    compiler_params=pltpu.CompilerParams(use_tc_tiling_on_sc=False),
    scratch_types=[
        pltpu.VMEM((R, 128), jnp.int32),
        pltpu.VMEM((R, 128), jnp.int32),
        pltpu.VMEM((5, 128, H), jnp.float32),
        pltpu.VMEM_SHARED((NP, H), jnp.float32),
        pltpu.SemaphoreType.DMA,
        pltpu.SemaphoreType.DMA,
        pltpu.SemaphoreType.DMA,
        pltpu.SemaphoreType.DMA,
        pltpu.SemaphoreType.DMA,
    ],
)
def _edge_kernel(table, srcs, dsts, zeros, out, idx_s, idx_d, buf, acc_sh,
                 gsem, ssem, gsem1, gsem2, gsem3):
    NB = 5
    W = 3
    c = lax.axis_index("c")
    s = lax.axis_index("s")
    rows0 = s * ROWS_T
    pltpu.sync_copy(srcs.at[c, pl.ds(s * R, R)], idx_s)
    pltpu.sync_copy(dsts.at[c, pl.ds(s * R, R)], idx_d)
    pltpu.sync_copy(zeros.at[pl.ds(rows0, ROWS_T)], acc_sh.at[pl.ds(rows0, ROWS_T)])
    plsc.subcore_barrier()

    sems = (gsem, gsem1, gsem2, gsem3)

    def body(g, carry):
        for k in range(4):
            @pl.when(g > 0)
            def _():
                pltpu.make_async_copy(table.at[idx_s.at[g - 4 + k]],
                                      buf.at[k], sems[k]).wait()
            pltpu.async_copy(table.at[idx_s.at[g + k]], buf.at[k], sems[k])
        return carry

    lax.fori_loop(0, R // 4, lambda g, c: body(g * 4, c), 0)
    for k in range(4):
        pltpu.make_async_copy(table.at[idx_s.at[R - 4 + k]], buf.at[k],
                              sems[k]).wait()
    plsc.subcore_barrier()
    pltpu.sync_copy(acc_sh.at[pl.ds(rows0, ROWS_T)], out.at[c, pl.ds(rows0, ROWS_T)])


def _enc_body(x_ref, w1_ref, b1_ref, w2_ref, b2_ref, w0_ref, deg_ref, y_ref):
    x = x_ref[0]
    h = jnp.maximum(jnp.dot(x, w1_ref[0], preferred_element_type=jnp.float32)
                    + b1_ref[0], 0.0)
    h = jnp.dot(h, w2_ref[0], preferred_element_type=jnp.float32) + b2_ref[0]
    dinv = lax.rsqrt(deg_ref[0, :, :1] + 1.0)
    y_ref[0] = jnp.dot(h, w0_ref[0], preferred_element_type=jnp.float32) * dinv


_enc_call = pl.pallas_call(
    _enc_body,
    grid=(2, NPB),
    in_specs=[
        pl.BlockSpec((1, BR, D), lambda c, i: (c, i, 0)),
        pl.BlockSpec((1, D, H), lambda c, i: (c, 0, 0)),
        pl.BlockSpec((1, 1, H), lambda c, i: (c, 0, 0)),
        pl.BlockSpec((1, H, H), lambda c, i: (c, 0, 0)),
        pl.BlockSpec((1, 1, H), lambda c, i: (c, 0, 0)),
        pl.BlockSpec((1, H, H), lambda c, i: (c, 0, 0)),
        pl.BlockSpec((1, BR, 8), lambda c, i: (c, i, 0)),
    ],
    out_specs=pl.BlockSpec((1, BR, H), lambda c, i: (c, i, 0)),
    out_shape=jax.ShapeDtypeStruct((2, NP, H), jnp.float32),
)


def _layer_body(acc_ref, y_ref, deg_ref, b_ref, w_ref, o_ref):
    dinv = lax.rsqrt(deg_ref[0, :, :1] + 1.0)
    h = jnp.maximum(dinv * (acc_ref[0] + y_ref[0]) + b_ref[0], 0.0)
    o_ref[0] = jnp.dot(h, w_ref[0], preferred_element_type=jnp.float32) * dinv


_layer_call = pl.pallas_call(
    _layer_body,
    grid=(2, NPB),
    in_specs=[
        pl.BlockSpec((1, BR, H), lambda c, i: (c, i, 0)),
        pl.BlockSpec((1, BR, H), lambda c, i: (c, i, 0)),
        pl.BlockSpec((1, BR, 8), lambda c, i: (c, i, 0)),
        pl.BlockSpec((1, 1, H), lambda c, i: (c, 0, 0)),
        pl.BlockSpec((1, H, H), lambda c, i: (c, 0, 0)),
    ],
    out_specs=pl.BlockSpec((1, BR, H), lambda c, i: (c, i, 0)),
    out_shape=jax.ShapeDtypeStruct((2, NP, H), jnp.float32),
)


def _final_body(acc_ref, y_ref, deg_ref, b_ref, w1_ref, b1_ref, w2_ref, b2_ref,
                wh_ref, bh_ref, o_ref):
    dinv = lax.rsqrt(deg_ref[:, :, :1] + 1.0)
    hf = jnp.maximum(dinv[0] * (acc_ref[0] + y_ref[0]) + b_ref[0], 0.0)
    hs = jnp.maximum(dinv[1] * (acc_ref[1] + y_ref[1]) + b_ref[1], 0.0)
    t = jnp.maximum(
        jnp.dot(hf, w1_ref[:H], preferred_element_type=jnp.float32)
        + jnp.dot(hs, w1_ref[H:], preferred_element_type=jnp.float32)
        + b1_ref[...], 0.0)
    u = jnp.dot(t, w2_ref[...], preferred_element_type=jnp.float32) + b2_ref[...]
    o_ref[...] = jnp.dot(u, wh_ref[...], preferred_element_type=jnp.float32) + bh_ref[...]


_final_call = pl.pallas_call(
    _final_body,
    grid=(NPB,),
    in_specs=[
        pl.BlockSpec((2, BR, H), lambda i: (0, i, 0)),
        pl.BlockSpec((2, BR, H), lambda i: (0, i, 0)),
        pl.BlockSpec((2, BR, 8), lambda i: (0, i, 0)),
        pl.BlockSpec((2, 1, H), lambda i: (0, 0, 0)),
        pl.BlockSpec((2 * H, H), lambda i: (0, 0)),
        pl.BlockSpec((1, H), lambda i: (0, 0)),
        pl.BlockSpec((H, H), lambda i: (0, 0)),
        pl.BlockSpec((1, H), lambda i: (0, 0)),
        pl.BlockSpec((H, H), lambda i: (0, 0)),
        pl.BlockSpec((1, H), lambda i: (0, 0)),
    ],
    out_specs=pl.BlockSpec((BR, H), lambda i: (i, 0)),
    out_shape=jax.ShapeDtypeStruct((NP, H), jnp.float32),
)


def _pad_rows(x, rows):
    return jnp.concatenate(
        [x, jnp.zeros((rows - x.shape[0],) + x.shape[1:], x.dtype)], axis=0)


def kernel(front_x, front_edge_index, front_edge_attr, side_x, side_edge_index,
           side_edge_attr, f_enc_w1, f_enc_b1, f_enc_w2, f_enc_b2, f_conv_w0,
           f_conv_b0, f_conv_w1, f_conv_b1, f_conv_w2, f_conv_b2, s_enc_w1,
           s_enc_b1, s_enc_w2, s_enc_b2, s_conv_w0, s_conv_b0, s_conv_w1,
           s_conv_b1, s_conv_w2, s_conv_b2, fus_w1, fus_b1, fus_w2, fus_b2,
           no_w, no_b, nt_w, nt_b):
    f32 = jnp.float32

    def prep_edges(ei):
        src = ei[0].astype(jnp.int32)
        dst = ei[1].astype(jnp.int32)
        src = jnp.concatenate([src, jnp.zeros((EPAD - E,), jnp.int32)])
        dst = jnp.concatenate([dst, jnp.full((EPAD - E,), N, jnp.int32)])
        return src, dst

    sf, df = prep_edges(front_edge_index)
    ss, ds2 = prep_edges(side_edge_index)
    srcs = jnp.stack([sf, ss + NP]).reshape(2, EPR, 128)
    dsts = jnp.stack([df, ds2]).reshape(2, EPR, 128)

    ones8 = jnp.tile(jnp.eye(1, 8, dtype=f32), (128, 1))
    zeros8 = jnp.zeros((NP, 8), f32)
    zerosH = jnp.zeros((NP, H), f32)

    deg = _deg_kernel(dsts, ones8, zeros8)

    x = jnp.stack([_pad_rows(front_x, NP), _pad_rows(side_x, NP)])
    ew1 = jnp.stack([f_enc_w1, s_enc_w1])
    eb1 = jnp.stack([f_enc_b1, s_enc_b1])[:, None, :]
    ew2 = jnp.stack([f_enc_w2, s_enc_w2])
    eb2 = jnp.stack([f_enc_b2, s_enc_b2])[:, None, :]
    cw = [jnp.stack([f_conv_w0, s_conv_w0]), jnp.stack([f_conv_w1, s_conv_w1]),
          jnp.stack([f_conv_w2, s_conv_w2])]
    cb = [jnp.stack([f_conv_b0, s_conv_b0])[:, None, :],
          jnp.stack([f_conv_b1, s_conv_b1])[:, None, :],
          jnp.stack([f_conv_b2, s_conv_b2])[:, None, :]]

    y = _enc_call(x, ew1, eb1, ew2, eb2, cw[0], deg)
    for i in range(3):
        acc = _edge_kernel(y.reshape(2 * NP, H), srcs, dsts, zerosH)
        if i < 2:
            y = _layer_call(acc, y, deg, cb[i], cw[i + 1])

    wh = jnp.zeros((H, H), f32).at[:, :32].set(no_w).at[:, 32:34].set(nt_w)
    bh = jnp.zeros((1, H), f32).at[0, :32].set(no_b).at[0, 32:34].set(nt_b)
    heads = _final_call(acc, y, deg, cb[2], fus_w1, fus_b1[None, :], fus_w2,
                        fus_b2[None, :], wh, bh)
    return heads[:N, :32], heads[:N, 32:34]


# speedup vs baseline: 1.9465x; 1.9465x over previous
import functools

import jax
import jax.numpy as jnp
from jax import lax
from jax.experimental import pallas as pl
from jax.experimental.pallas import tpu as pltpu
from jax.experimental.pallas import tpu_sc as plsc

N = 10000
E = 320000
D = 128
H = 64

NT = 16
NP = 10240
ROWS_T = NP // NT
EPR = 2560
EPAD = EPR * 128
R = EPR // NT

BR = 1024
NPB = NP // BR

_mesh = plsc.VectorSubcoreMesh(core_axis_name="c", subcore_axis_name="s")


@functools.partial(
    pl.kernel,
    out_type=jax.ShapeDtypeStruct((2, NP, 8), jnp.float32),
    mesh=_mesh,
    compiler_params=pltpu.CompilerParams(use_tc_tiling_on_sc=False),
    scratch_types=[
        pltpu.VMEM((R, 128), jnp.int32),
        pltpu.VMEM((128, 8), jnp.float32),
        pltpu.VMEM_SHARED((NP, 8), jnp.float32),
        pltpu.SemaphoreType.DMA,
    ],
)
def _deg_kernel(dsts, ones8, zeros8, out, idx_v, ones_v, acc_sh, sem):
    c = lax.axis_index("c")
    s = lax.axis_index("s")
    rows0 = s * ROWS_T
    pltpu.sync_copy(dsts.at[c, pl.ds(s * R, R)], idx_v)
    pltpu.sync_copy(ones8, ones_v)
    pltpu.sync_copy(zeros8.at[pl.ds(rows0, ROWS_T)], acc_sh.at[pl.ds(rows0, ROWS_T)])
    plsc.subcore_barrier()

    def fire(j, carry):
        pltpu.async_copy(ones_v, acc_sh.at[idx_v.at[j]], sem, add=True)
        return carry

    lax.fori_loop(0, R, fire, 0)

    def drain(j, carry):
        pltpu.make_async_copy(ones_v, acc_sh.at[idx_v.at[j]], sem).wait()
        return carry

    lax.fori_loop(0, R, drain, 0)
    plsc.subcore_barrier()
    pltpu.sync_copy(acc_sh.at[pl.ds(rows0, ROWS_T)], out.at[c, pl.ds(rows0, ROWS_T)])


@functools.partial(
    pl.kernel,
    out_type=jax.ShapeDtypeStruct((2, NP, H), jnp.float32),
    mesh=_mesh,
    compiler_params=pltpu.CompilerParams(use_tc_tiling_on_sc=False),
    scratch_types=[
        pltpu.VMEM((2, R // 4, 128), jnp.int32),
        pltpu.VMEM((2, R // 4, 128), jnp.int32),
        pltpu.VMEM((3, 128, H), jnp.float32),
        pltpu.VMEM_SHARED((NP, H), jnp.float32),
        pltpu.VMEM_SHARED((NP, H), jnp.float32),
        pltpu.SemaphoreType.DMA,
        pltpu.SemaphoreType.DMA,
        pltpu.SemaphoreType.DMA,
    ],
)
def _edge_kernel(y_hbm, srcs, dsts, zeros, out, idx_s, idx_d, buf, y_sh,
                 acc_sh, gsem, ssem, isem):
    QR = R // 4
    c = lax.axis_index("c")
    s = lax.axis_index("s")
    rows0 = s * ROWS_T
    er0 = s * R
    pltpu.sync_copy(srcs.at[c, pl.ds(er0, QR)], idx_s.at[0])
    pltpu.sync_copy(dsts.at[c, pl.ds(er0, QR)], idx_d.at[0])
    pltpu.sync_copy(y_hbm.at[c, pl.ds(rows0, ROWS_T)],
                    y_sh.at[pl.ds(rows0, ROWS_T)])
    pltpu.sync_copy(zeros.at[pl.ds(rows0, ROWS_T)], acc_sh.at[pl.ds(rows0, ROWS_T)])
    plsc.subcore_barrier()

    for ph in range(4):
        sl = ph % 2
        if ph < 3:
            nxt = er0 + (ph + 1) * QR
            ip_s = pltpu.async_copy(srcs.at[c, pl.ds(nxt, QR)],
                                    idx_s.at[1 - sl], isem)
            ip_d = pltpu.async_copy(dsts.at[c, pl.ds(nxt, QR)],
                                    idx_d.at[1 - sl], isem)

        for k in range(2):
            pltpu.async_copy(y_sh.at[idx_s.at[sl, k]], buf.at[k], gsem)

        def body(j, carry):
            p = lax.rem(j, 3)
            pltpu.make_async_copy(y_sh.at[idx_s.at[sl, j]], buf.at[p],
                                  gsem).wait()
            pltpu.async_copy(buf.at[p], acc_sh.at[idx_d.at[sl, j]], ssem,
                             add=True)

            @pl.when(j + 2 < QR)
            def _():
                q = lax.rem(j + 2, 3)

                @pl.when(j >= 1)
                def _():
                    pltpu.make_async_copy(buf.at[q],
                                          acc_sh.at[idx_d.at[sl, j - 1]],
                                          ssem).wait()

                pltpu.async_copy(y_sh.at[idx_s.at[sl, j + 2]], buf.at[q], gsem)

            return carry

        lax.fori_loop(0, QR, body, 0, unroll=2)

        def drain(j, carry):
            pltpu.make_async_copy(buf.at[lax.rem(j, 3)],
                                  acc_sh.at[idx_d.at[sl, j]], ssem).wait()
            return carry

        lax.fori_loop(QR - 3, QR, drain, 0)
        if ph < 3:
            ip_s.wait()
            ip_d.wait()
    plsc.subcore_barrier()
    pltpu.sync_copy(acc_sh.at[pl.ds(rows0, ROWS_T)], out.at[c, pl.ds(rows0, ROWS_T)])


def _enc_body(x_ref, w1_ref, b1_ref, w2_ref, b2_ref, w0_ref, deg_ref, y_ref):
    x = x_ref[0]
    h = jnp.maximum(jnp.dot(x, w1_ref[0], preferred_element_type=jnp.float32)
                    + b1_ref[0], 0.0)
    h = jnp.dot(h, w2_ref[0], preferred_element_type=jnp.float32) + b2_ref[0]
    dinv = lax.rsqrt(deg_ref[0, :, :1] + 1.0)
    y_ref[0] = jnp.dot(h, w0_ref[0], preferred_element_type=jnp.float32) * dinv


_enc_call = pl.pallas_call(
    _enc_body,
    grid=(2, NPB),
    in_specs=[
        pl.BlockSpec((1, BR, D), lambda c, i: (c, i, 0)),
        pl.BlockSpec((1, D, H), lambda c, i: (c, 0, 0)),
        pl.BlockSpec((1, 1, H), lambda c, i: (c, 0, 0)),
        pl.BlockSpec((1, H, H), lambda c, i: (c, 0, 0)),
        pl.BlockSpec((1, 1, H), lambda c, i: (c, 0, 0)),
        pl.BlockSpec((1, H, H), lambda c, i: (c, 0, 0)),
        pl.BlockSpec((1, BR, 8), lambda c, i: (c, i, 0)),
    ],
    out_specs=pl.BlockSpec((1, BR, H), lambda c, i: (c, i, 0)),
    out_shape=jax.ShapeDtypeStruct((2, NP, H), jnp.float32),
)


def _layer_body(acc_ref, y_ref, deg_ref, b_ref, w_ref, o_ref):
    dinv = lax.rsqrt(deg_ref[0, :, :1] + 1.0)
    h = jnp.maximum(dinv * (acc_ref[0] + y_ref[0]) + b_ref[0], 0.0)
    o_ref[0] = jnp.dot(h, w_ref[0], preferred_element_type=jnp.float32) * dinv


_layer_call = pl.pallas_call(
    _layer_body,
    grid=(2, NPB),
    in_specs=[
        pl.BlockSpec((1, BR, H), lambda c, i: (c, i, 0)),
        pl.BlockSpec((1, BR, H), lambda c, i: (c, i, 0)),
        pl.BlockSpec((1, BR, 8), lambda c, i: (c, i, 0)),
        pl.BlockSpec((1, 1, H), lambda c, i: (c, 0, 0)),
        pl.BlockSpec((1, H, H), lambda c, i: (c, 0, 0)),
    ],
    out_specs=pl.BlockSpec((1, BR, H), lambda c, i: (c, i, 0)),
    out_shape=jax.ShapeDtypeStruct((2, NP, H), jnp.float32),
)


def _final_body(acc_ref, y_ref, deg_ref, b_ref, w1_ref, b1_ref, w2_ref, b2_ref,
                wh_ref, bh_ref, o_ref):
    dinv = lax.rsqrt(deg_ref[:, :, :1] + 1.0)
    hf = jnp.maximum(dinv[0] * (acc_ref[0] + y_ref[0]) + b_ref[0], 0.0)
    hs = jnp.maximum(dinv[1] * (acc_ref[1] + y_ref[1]) + b_ref[1], 0.0)
    t = jnp.maximum(
        jnp.dot(hf, w1_ref[:H], preferred_element_type=jnp.float32)
        + jnp.dot(hs, w1_ref[H:], preferred_element_type=jnp.float32)
        + b1_ref[...], 0.0)
    u = jnp.dot(t, w2_ref[...], preferred_element_type=jnp.float32) + b2_ref[...]
    o_ref[...] = jnp.dot(u, wh_ref[...], preferred_element_type=jnp.float32) + bh_ref[...]


_final_call = pl.pallas_call(
    _final_body,
    grid=(NPB,),
    in_specs=[
        pl.BlockSpec((2, BR, H), lambda i: (0, i, 0)),
        pl.BlockSpec((2, BR, H), lambda i: (0, i, 0)),
        pl.BlockSpec((2, BR, 8), lambda i: (0, i, 0)),
        pl.BlockSpec((2, 1, H), lambda i: (0, 0, 0)),
        pl.BlockSpec((2 * H, H), lambda i: (0, 0)),
        pl.BlockSpec((1, H), lambda i: (0, 0)),
        pl.BlockSpec((H, H), lambda i: (0, 0)),
        pl.BlockSpec((1, H), lambda i: (0, 0)),
        pl.BlockSpec((H, H), lambda i: (0, 0)),
        pl.BlockSpec((1, H), lambda i: (0, 0)),
    ],
    out_specs=pl.BlockSpec((BR, H), lambda i: (i, 0)),
    out_shape=jax.ShapeDtypeStruct((NP, H), jnp.float32),
)


def _pad_rows(x, rows):
    return jnp.concatenate(
        [x, jnp.zeros((rows - x.shape[0],) + x.shape[1:], x.dtype)], axis=0)


def kernel(front_x, front_edge_index, front_edge_attr, side_x, side_edge_index,
           side_edge_attr, f_enc_w1, f_enc_b1, f_enc_w2, f_enc_b2, f_conv_w0,
           f_conv_b0, f_conv_w1, f_conv_b1, f_conv_w2, f_conv_b2, s_enc_w1,
           s_enc_b1, s_enc_w2, s_enc_b2, s_conv_w0, s_conv_b0, s_conv_w1,
           s_conv_b1, s_conv_w2, s_conv_b2, fus_w1, fus_b1, fus_w2, fus_b2,
           no_w, no_b, nt_w, nt_b):
    f32 = jnp.float32

    def prep_edges(ei):
        src = ei[0].astype(jnp.int32)
        dst = ei[1].astype(jnp.int32)
        src = jnp.concatenate([src, jnp.zeros((EPAD - E,), jnp.int32)])
        dst = jnp.concatenate([dst, jnp.full((EPAD - E,), N, jnp.int32)])
        return src, dst

    sf, df = prep_edges(front_edge_index)
    ss, ds2 = prep_edges(side_edge_index)
    srcs = jnp.stack([sf, ss]).reshape(2, EPR, 128)
    dsts = jnp.stack([df, ds2]).reshape(2, EPR, 128)

    ones8 = jnp.tile(jnp.eye(1, 8, dtype=f32), (128, 1))
    zeros8 = jnp.zeros((NP, 8), f32)
    zerosH = jnp.zeros((NP, H), f32)

    deg = _deg_kernel(dsts, ones8, zeros8)

    x = jnp.stack([_pad_rows(front_x, NP), _pad_rows(side_x, NP)])
    ew1 = jnp.stack([f_enc_w1, s_enc_w1])
    eb1 = jnp.stack([f_enc_b1, s_enc_b1])[:, None, :]
    ew2 = jnp.stack([f_enc_w2, s_enc_w2])
    eb2 = jnp.stack([f_enc_b2, s_enc_b2])[:, None, :]
    cw = [jnp.stack([f_conv_w0, s_conv_w0]), jnp.stack([f_conv_w1, s_conv_w1]),
          jnp.stack([f_conv_w2, s_conv_w2])]
    cb = [jnp.stack([f_conv_b0, s_conv_b0])[:, None, :],
          jnp.stack([f_conv_b1, s_conv_b1])[:, None, :],
          jnp.stack([f_conv_b2, s_conv_b2])[:, None, :]]

    y = _enc_call(x, ew1, eb1, ew2, eb2, cw[0], deg)
    for i in range(3):
        acc = _edge_kernel(y, srcs, dsts, zerosH)
        if i < 2:
            y = _layer_call(acc, y, deg, cb[i], cw[i + 1])

    wh = jnp.zeros((H, H), f32).at[:, :32].set(no_w).at[:, 32:34].set(nt_w)
    bh = jnp.zeros((1, H), f32).at[0, :32].set(no_b).at[0, 32:34].set(nt_b)
    heads = _final_call(acc, y, deg, cb[2], fus_w1, fus_b1[None, :], fus_w2,
                        fus_b2[None, :], wh, bh)
    return heads[:N, :32], heads[:N, 32:34]


# speedup vs baseline: 2.1717x; 1.1157x over previous
import functools

import jax
import jax.numpy as jnp
from jax import lax
from jax.experimental import pallas as pl
from jax.experimental.pallas import tpu as pltpu
from jax.experimental.pallas import tpu_sc as plsc

N = 10000
E = 320000
D = 128
H = 64

NT = 16
NP = 10240
ROWS_T = NP // NT
EPR = 2560
EPAD = EPR * 128
R = EPR // NT

PR = NP // 2
BR2 = 1024
NPB2 = PR // BR2

_mesh = plsc.VectorSubcoreMesh(core_axis_name="c", subcore_axis_name="s")


@functools.partial(
    pl.kernel,
    out_type=jax.ShapeDtypeStruct((2, NP, 8), jnp.float32),
    mesh=_mesh,
    compiler_params=pltpu.CompilerParams(use_tc_tiling_on_sc=False),
    scratch_types=[
        pltpu.VMEM((R, 128), jnp.int32),
        pltpu.VMEM((128, 8), jnp.float32),
        pltpu.VMEM_SHARED((NP, 8), jnp.float32),
        pltpu.SemaphoreType.DMA,
    ],
)
def _deg_kernel(dsts, ones8, zeros8, out, idx_v, ones_v, acc_sh, sem):
    c = lax.axis_index("c")
    s = lax.axis_index("s")
    rows0 = s * ROWS_T
    pltpu.sync_copy(dsts.at[c, pl.ds(s * R, R)], idx_v)
    pltpu.sync_copy(ones8, ones_v)
    pltpu.sync_copy(zeros8.at[pl.ds(rows0, ROWS_T)], acc_sh.at[pl.ds(rows0, ROWS_T)])
    plsc.subcore_barrier()

    def fire(j, carry):
        pltpu.async_copy(ones_v, acc_sh.at[idx_v.at[j]], sem, add=True)
        return carry

    lax.fori_loop(0, R, fire, 0)

    def drain(j, carry):
        pltpu.make_async_copy(ones_v, acc_sh.at[idx_v.at[j]], sem).wait()
        return carry

    lax.fori_loop(0, R, drain, 0)
    plsc.subcore_barrier()
    pltpu.sync_copy(acc_sh.at[pl.ds(rows0, ROWS_T)], out.at[c, pl.ds(rows0, ROWS_T)])


@functools.partial(
    pl.kernel,
    out_type=jax.ShapeDtypeStruct((2, NP, H), jnp.float32),
    mesh=_mesh,
    compiler_params=pltpu.CompilerParams(use_tc_tiling_on_sc=False),
    scratch_types=[
        pltpu.VMEM((2, R // 4, 128), jnp.int32),
        pltpu.VMEM((2, R // 4, 128), jnp.int32),
        pltpu.VMEM((3, 128, H), jnp.float32),
        pltpu.VMEM_SHARED((NP, H), jnp.float32),
        pltpu.VMEM_SHARED((NP, H), jnp.float32),
        pltpu.SemaphoreType.DMA,
        pltpu.SemaphoreType.DMA,
        pltpu.SemaphoreType.DMA,
    ],
)
def _edge_kernel(y_hbm, srcs, dsts, zeros, out, idx_s, idx_d, buf, y_sh,
                 acc_sh, gsem, ssem, isem):
    QR = R // 4
    c = lax.axis_index("c")
    s = lax.axis_index("s")
    rows0 = s * ROWS_T
    er0 = s * R
    pltpu.sync_copy(srcs.at[c, pl.ds(er0, QR)], idx_s.at[0])
    pltpu.sync_copy(dsts.at[c, pl.ds(er0, QR)], idx_d.at[0])
    pltpu.sync_copy(y_hbm.at[c, pl.ds(rows0, ROWS_T)],
                    y_sh.at[pl.ds(rows0, ROWS_T)])
    pltpu.sync_copy(zeros.at[pl.ds(rows0, ROWS_T)], acc_sh.at[pl.ds(rows0, ROWS_T)])
    plsc.subcore_barrier()

    for ph in range(4):
        sl = ph % 2
        if ph < 3:
            nxt = er0 + (ph + 1) * QR
            ip_s = pltpu.async_copy(srcs.at[c, pl.ds(nxt, QR)],
                                    idx_s.at[1 - sl], isem)
            ip_d = pltpu.async_copy(dsts.at[c, pl.ds(nxt, QR)],
                                    idx_d.at[1 - sl], isem)

        for k in range(2):
            pltpu.async_copy(y_sh.at[idx_s.at[sl, k]], buf.at[k], gsem)

        def body(j, carry):
            p = lax.rem(j, 3)
            pltpu.make_async_copy(y_sh.at[idx_s.at[sl, j]], buf.at[p],
                                  gsem).wait()
            pltpu.async_copy(buf.at[p], acc_sh.at[idx_d.at[sl, j]], ssem,
                             add=True)

            @pl.when(j + 2 < QR)
            def _():
                q = lax.rem(j + 2, 3)

                @pl.when(j >= 1)
                def _():
                    pltpu.make_async_copy(buf.at[q],
                                          acc_sh.at[idx_d.at[sl, j - 1]],
                                          ssem).wait()

                pltpu.async_copy(y_sh.at[idx_s.at[sl, j + 2]], buf.at[q], gsem)

            return carry

        lax.fori_loop(0, QR, body, 0, unroll=2)

        def drain(j, carry):
            pltpu.make_async_copy(buf.at[lax.rem(j, 3)],
                                  acc_sh.at[idx_d.at[sl, j]], ssem).wait()
            return carry

        lax.fori_loop(QR - 3, QR, drain, 0)
        if ph < 3:
            ip_s.wait()
            ip_d.wait()
    plsc.subcore_barrier()
    pltpu.sync_copy(acc_sh.at[pl.ds(rows0, ROWS_T)], out.at[c, pl.ds(rows0, ROWS_T)])


def _enc_body(x_ref, w1_ref, b1_ref, w2_ref, b2_ref, w0_ref, deg_ref, y_ref):
    x = x_ref[0]
    h = jnp.maximum(jnp.dot(x, w1_ref[0], preferred_element_type=jnp.float32)
                    + b1_ref[0], 0.0)
    h = jnp.dot(h, w2_ref[0], preferred_element_type=jnp.float32) + b2_ref[0]
    dinv = lax.rsqrt(deg_ref[0] + 1.0)
    y_ref[0] = jnp.dot(h, w0_ref[0], preferred_element_type=jnp.float32) * dinv


_enc_call = pl.pallas_call(
    _enc_body,
    grid=(2, NPB2),
    in_specs=[
        pl.BlockSpec((1, BR2, 2 * D), lambda c, i: (c, i, 0)),
        pl.BlockSpec((1, 2 * D, 2 * H), lambda c, i: (c, 0, 0)),
        pl.BlockSpec((1, 1, 2 * H), lambda c, i: (c, 0, 0)),
        pl.BlockSpec((1, 2 * H, 2 * H), lambda c, i: (c, 0, 0)),
        pl.BlockSpec((1, 1, 2 * H), lambda c, i: (c, 0, 0)),
        pl.BlockSpec((1, 2 * H, 2 * H), lambda c, i: (c, 0, 0)),
        pl.BlockSpec((1, BR2, 2 * H), lambda c, i: (c, i, 0)),
    ],
    out_specs=pl.BlockSpec((1, BR2, 2 * H), lambda c, i: (c, i, 0)),
    out_shape=jax.ShapeDtypeStruct((2, PR, 2 * H), jnp.float32),
)


def _layer_body(acc_ref, y_ref, deg_ref, b_ref, w_ref, o_ref):
    dinv = lax.rsqrt(deg_ref[0] + 1.0)
    h = jnp.maximum(dinv * (acc_ref[0] + y_ref[0]) + b_ref[0], 0.0)
    o_ref[0] = jnp.dot(h, w_ref[0], preferred_element_type=jnp.float32) * dinv


_layer_call = pl.pallas_call(
    _layer_body,
    grid=(2, NPB2),
    in_specs=[
        pl.BlockSpec((1, BR2, 2 * H), lambda c, i: (c, i, 0)),
        pl.BlockSpec((1, BR2, 2 * H), lambda c, i: (c, i, 0)),
        pl.BlockSpec((1, BR2, 2 * H), lambda c, i: (c, i, 0)),
        pl.BlockSpec((1, 1, 2 * H), lambda c, i: (c, 0, 0)),
        pl.BlockSpec((1, 2 * H, 2 * H), lambda c, i: (c, 0, 0)),
    ],
    out_specs=pl.BlockSpec((1, BR2, 2 * H), lambda c, i: (c, i, 0)),
    out_shape=jax.ShapeDtypeStruct((2, PR, 2 * H), jnp.float32),
)


def _final_body(acc_ref, y_ref, deg_ref, b_ref, w1a_ref, w1b_ref, b1_ref,
                w2_ref, b2_ref, wh_ref, bh_ref, o_ref):
    dinv = lax.rsqrt(deg_ref[...] + 1.0)
    hf = jnp.maximum(dinv[0] * (acc_ref[0] + y_ref[0]) + b_ref[0], 0.0)
    hs = jnp.maximum(dinv[1] * (acc_ref[1] + y_ref[1]) + b_ref[1], 0.0)
    t = jnp.maximum(
        jnp.dot(hf, w1a_ref[...], preferred_element_type=jnp.float32)
        + jnp.dot(hs, w1b_ref[...], preferred_element_type=jnp.float32)
        + b1_ref[...], 0.0)
    u = jnp.dot(t, w2_ref[...], preferred_element_type=jnp.float32) + b2_ref[...]
    o_ref[...] = jnp.dot(u, wh_ref[...], preferred_element_type=jnp.float32) + bh_ref[...]


_final_call = pl.pallas_call(
    _final_body,
    grid=(NPB2,),
    in_specs=[
        pl.BlockSpec((2, BR2, 2 * H), lambda i: (0, i, 0)),
        pl.BlockSpec((2, BR2, 2 * H), lambda i: (0, i, 0)),
        pl.BlockSpec((2, BR2, 2 * H), lambda i: (0, i, 0)),
        pl.BlockSpec((2, 1, 2 * H), lambda i: (0, 0, 0)),
        pl.BlockSpec((2 * H, 2 * H), lambda i: (0, 0)),
        pl.BlockSpec((2 * H, 2 * H), lambda i: (0, 0)),
        pl.BlockSpec((1, 2 * H), lambda i: (0, 0)),
        pl.BlockSpec((2 * H, 2 * H), lambda i: (0, 0)),
        pl.BlockSpec((1, 2 * H), lambda i: (0, 0)),
        pl.BlockSpec((2 * H, 2 * H), lambda i: (0, 0)),
        pl.BlockSpec((1, 2 * H), lambda i: (0, 0)),
    ],
    out_specs=pl.BlockSpec((BR2, 2 * H), lambda i: (i, 0)),
    out_shape=jax.ShapeDtypeStruct((PR, 2 * H), jnp.float32),
)


def _pad_rows(x, rows):
    return jnp.concatenate(
        [x, jnp.zeros((rows - x.shape[0],) + x.shape[1:], x.dtype)], axis=0)


def _bd(w):
    z = jnp.zeros_like(w)
    return jnp.concatenate(
        [jnp.concatenate([w, z], 1), jnp.concatenate([z, w], 1)], 0)


def _tile2(b):
    return jnp.concatenate([b, b])


def kernel(front_x, front_edge_index, front_edge_attr, side_x, side_edge_index,
           side_edge_attr, f_enc_w1, f_enc_b1, f_enc_w2, f_enc_b2, f_conv_w0,
           f_conv_b0, f_conv_w1, f_conv_b1, f_conv_w2, f_conv_b2, s_enc_w1,
           s_enc_b1, s_enc_w2, s_enc_b2, s_conv_w0, s_conv_b0, s_conv_w1,
           s_conv_b1, s_conv_w2, s_conv_b2, fus_w1, fus_b1, fus_w2, fus_b2,
           no_w, no_b, nt_w, nt_b):
    f32 = jnp.float32

    def prep_edges(ei):
        src = ei[0].astype(jnp.int32)
        dst = ei[1].astype(jnp.int32)
        src = jnp.concatenate([src, jnp.zeros((EPAD - E,), jnp.int32)])
        dst = jnp.concatenate([dst, jnp.full((EPAD - E,), N, jnp.int32)])
        return src, dst

    sf, df = prep_edges(front_edge_index)
    ss, ds2 = prep_edges(side_edge_index)
    srcs = jnp.stack([sf, ss]).reshape(2, EPR, 128)
    dsts = jnp.stack([df, ds2]).reshape(2, EPR, 128)

    ones8 = jnp.tile(jnp.eye(1, 8, dtype=f32), (128, 1))
    zeros8 = jnp.zeros((NP, 8), f32)
    zerosH = jnp.zeros((NP, H), f32)

    deg = _deg_kernel(dsts, ones8, zeros8)
    degp = jnp.broadcast_to(deg[..., :1].reshape(2, PR, 2, 1),
                            (2, PR, 2, H)).reshape(2, PR, 2 * H)

    xp = jnp.stack([_pad_rows(front_x, NP),
                    _pad_rows(side_x, NP)]).reshape(2, PR, 2 * D)
    ew1 = jnp.stack([_bd(f_enc_w1), _bd(s_enc_w1)])
    eb1 = jnp.stack([_tile2(f_enc_b1), _tile2(s_enc_b1)])[:, None, :]
    ew2 = jnp.stack([_bd(f_enc_w2), _bd(s_enc_w2)])
    eb2 = jnp.stack([_tile2(f_enc_b2), _tile2(s_enc_b2)])[:, None, :]
    cw = [jnp.stack([_bd(f_conv_w0), _bd(s_conv_w0)]),
          jnp.stack([_bd(f_conv_w1), _bd(s_conv_w1)]),
          jnp.stack([_bd(f_conv_w2), _bd(s_conv_w2)])]
    cb = [jnp.stack([_tile2(f_conv_b0), _tile2(s_conv_b0)])[:, None, :],
          jnp.stack([_tile2(f_conv_b1), _tile2(s_conv_b1)])[:, None, :],
          jnp.stack([_tile2(f_conv_b2), _tile2(s_conv_b2)])[:, None, :]]

    y = _enc_call(xp, ew1, eb1, ew2, eb2, cw[0], degp)
    accp = None
    for i in range(3):
        acc = _edge_kernel(y.reshape(2, NP, H), srcs, dsts, zerosH)
        accp = acc.reshape(2, PR, 2 * H)
        if i < 2:
            y = _layer_call(accp, y, degp, cb[i], cw[i + 1])

    wh = jnp.zeros((H, H), f32).at[:, :32].set(no_w).at[:, 32:34].set(nt_w)
    bh = jnp.zeros((H,), f32).at[:32].set(no_b).at[32:34].set(nt_b)
    heads_p = _final_call(accp, y, degp, cb[2], _bd(fus_w1[:H]),
                          _bd(fus_w1[H:]), _tile2(fus_b1)[None, :],
                          _bd(fus_w2), _tile2(fus_b2)[None, :], _bd(wh),
                          _tile2(bh)[None, :])
    heads = heads_p.reshape(NP, H)
    return heads[:N, :32], heads[:N, 32:34]


# speedup vs baseline: 2.4937x; 1.1483x over previous
import functools

import jax
import jax.numpy as jnp
from jax import lax
from jax.experimental import pallas as pl
from jax.experimental.pallas import tpu as pltpu
from jax.experimental.pallas import tpu_sc as plsc

N = 10000
E = 320000
D = 128
H = 64

NT = 16
NP = 10240
ROWS_T = NP // NT
ERAW = E // 128
ERW = 2516
QR = 40

PR = NP // 2
BR2 = 1024
NPB2 = PR // BR2

_mesh = plsc.VectorSubcoreMesh(core_axis_name="c", subcore_axis_name="s")


@functools.partial(
    pl.kernel,
    out_type=jax.ShapeDtypeStruct((2, PR * 2 * H), jnp.float32),
    mesh=_mesh,
    compiler_params=pltpu.CompilerParams(use_tc_tiling_on_sc=False),
    scratch_types=[
        pltpu.VMEM((160, 128), jnp.int32),
        pltpu.VMEM((128,), jnp.float32),
        pltpu.VMEM((ROWS_T,), jnp.float32),
        pltpu.VMEM((ROWS_T * H,), jnp.float32),
        pltpu.VMEM_SHARED((NP,), jnp.float32),
        pltpu.SemaphoreType.DMA,
    ],
)
def _deg_kernel(fe, se, ones128, zeros1, out, idx_v, ones_v, dbuf, obuf,
                acc_sh, sem):
    c = lax.axis_index("c")
    s = lax.axis_index("s")
    rows0 = s * ROWS_T
    st = 156 * s + jnp.maximum(s - 12, 0)
    w = 156 + (s >= 12).astype(jnp.int32)

    @pl.when(c == 0)
    def _():
        pltpu.sync_copy(fe.at[1, pl.ds(st, 160)], idx_v)

    @pl.when(c == 1)
    def _():
        pltpu.sync_copy(se.at[1, pl.ds(st, 160)], idx_v)

    pltpu.sync_copy(ones128, ones_v)
    pltpu.sync_copy(zeros1.at[pl.ds(rows0, ROWS_T)], acc_sh.at[pl.ds(rows0, ROWS_T)])
    plsc.subcore_barrier()

    def fire(j, carry):
        pltpu.async_copy(ones_v, acc_sh.at[idx_v.at[j]], sem, add=True)
        return carry

    lax.fori_loop(0, w, fire, 0)

    def drain(j, carry):
        pltpu.make_async_copy(ones_v, acc_sh.at[idx_v.at[j]], sem).wait()
        return carry

    lax.fori_loop(0, w, drain, 0)
    plsc.subcore_barrier()

    pltpu.sync_copy(acc_sh.at[pl.ds(rows0, ROWS_T)], dbuf)

    def expand(g, carry):
        v = dbuf[pl.ds(16 * g, 16)]
        for j in range(16):
            splat = lax.gather(
                v, jnp.full((16, 1), j, jnp.int32),
                lax.GatherDimensionNumbers(offset_dims=(),
                                           collapsed_slice_dims=(0,),
                                           start_index_map=(0,)),
                (1,), mode=lax.GatherScatterMode.PROMISE_IN_BOUNDS)
            for m in range(4):
                obuf[pl.ds(1024 * g + 64 * j + 16 * m, 16)] = splat
        return carry

    lax.fori_loop(0, ROWS_T // 16, expand, 0)
    pltpu.sync_copy(obuf, out.at[c, pl.ds(s * (ROWS_T * H), ROWS_T * H)])


@functools.partial(
    pl.kernel,
    out_type=jax.ShapeDtypeStruct((2, NP, H), jnp.float32),
    mesh=_mesh,
    compiler_params=pltpu.CompilerParams(use_tc_tiling_on_sc=False),
    scratch_types=[
        pltpu.VMEM((2, QR, 128), jnp.int32),
        pltpu.VMEM((2, QR, 128), jnp.int32),
        pltpu.VMEM((3, 128, H), jnp.float32),
        pltpu.VMEM_SHARED((NP, H), jnp.float32),
        pltpu.VMEM_SHARED((NP, H), jnp.float32),
        pltpu.SemaphoreType.DMA,
        pltpu.SemaphoreType.DMA,
        pltpu.SemaphoreType.DMA,
    ],
)
def _edge_kernel(y_hbm, fe, se, zeros, out, idx_s, idx_d, buf, y_sh,
                 acc_sh, gsem, ssem, isem):
    c = lax.axis_index("c")
    s = lax.axis_index("s")
    rows0 = s * ROWS_T
    st = 156 * s + jnp.maximum(s - 12, 0)
    w = 156 + (s >= 12).astype(jnp.int32)

    def stage(off, sl):
        @pl.when(c == 0)
        def _():
            pltpu.async_copy(fe.at[0, pl.ds(off, QR)], idx_s.at[sl], isem)
            pltpu.async_copy(fe.at[1, pl.ds(off, QR)], idx_d.at[sl], isem)

        @pl.when(c == 1)
        def _():
            pltpu.async_copy(se.at[0, pl.ds(off, QR)], idx_s.at[sl], isem)
            pltpu.async_copy(se.at[1, pl.ds(off, QR)], idx_d.at[sl], isem)

    def stage_wait(off, sl):
        @pl.when(c == 0)
        def _():
            pltpu.make_async_copy(fe.at[0, pl.ds(off, QR)], idx_s.at[sl], isem).wait()
            pltpu.make_async_copy(fe.at[1, pl.ds(off, QR)], idx_d.at[sl], isem).wait()

        @pl.when(c == 1)
        def _():
            pltpu.make_async_copy(se.at[0, pl.ds(off, QR)], idx_s.at[sl], isem).wait()
            pltpu.make_async_copy(se.at[1, pl.ds(off, QR)], idx_d.at[sl], isem).wait()

    stage(st, 0)
    pltpu.sync_copy(y_hbm.at[c, pl.ds(rows0, ROWS_T)],
                    y_sh.at[pl.ds(rows0, ROWS_T)])
    pltpu.sync_copy(zeros.at[pl.ds(rows0, ROWS_T)], acc_sh.at[pl.ds(rows0, ROWS_T)])
    stage_wait(st, 0)
    plsc.subcore_barrier()

    for ph in range(4):
        sl = ph % 2
        if ph < 3:
            stage(st + (ph + 1) * QR, 1 - sl)
        nb = QR if ph < 3 else w - 3 * QR

        for k in range(2):
            pltpu.async_copy(y_sh.at[idx_s.at[sl, k]], buf.at[k], gsem)

        def body(j, carry):
            p = lax.rem(j, 3)
            pltpu.make_async_copy(y_sh.at[idx_s.at[sl, j]], buf.at[p],
                                  gsem).wait()
            pltpu.async_copy(buf.at[p], acc_sh.at[idx_d.at[sl, j]], ssem,
                             add=True)

            @pl.when(j + 2 < nb)
            def _():
                q = lax.rem(j + 2, 3)

                @pl.when(j >= 1)
                def _():
                    pltpu.make_async_copy(buf.at[q],
                                          acc_sh.at[idx_d.at[sl, j - 1]],
                                          ssem).wait()

                pltpu.async_copy(y_sh.at[idx_s.at[sl, j + 2]], buf.at[q], gsem)

            return carry

        lax.fori_loop(0, nb, body, 0)

        def drain(j, carry):
            pltpu.make_async_copy(buf.at[lax.rem(j, 3)],
                                  acc_sh.at[idx_d.at[sl, j]], ssem).wait()
            return carry

        lax.fori_loop(nb - 3, nb, drain, 0)
        if ph < 3:
            stage_wait(st + (ph + 1) * QR, 1 - sl)
    plsc.subcore_barrier()
    pltpu.sync_copy(acc_sh.at[pl.ds(rows0, ROWS_T)], out.at[c, pl.ds(rows0, ROWS_T)])


def _enc_body(x_ref, w1_ref, b1_ref, w2_ref, b2_ref, w0_ref, deg_ref, y_ref):
    x = x_ref[0]
    h = jnp.maximum(jnp.dot(x, w1_ref[0], preferred_element_type=jnp.float32)
                    + b1_ref[0], 0.0)
    h = jnp.dot(h, w2_ref[0], preferred_element_type=jnp.float32) + b2_ref[0]
    dinv = lax.rsqrt(deg_ref[0] + 1.0)
    y_ref[0] = jnp.dot(h, w0_ref[0], preferred_element_type=jnp.float32) * dinv


_enc_call = pl.pallas_call(
    _enc_body,
    grid=(2, NPB2),
    in_specs=[
        pl.BlockSpec((1, BR2, 2 * D), lambda c, i: (c, i, 0)),
        pl.BlockSpec((1, 2 * D, 2 * H), lambda c, i: (c, 0, 0)),
        pl.BlockSpec((1, 1, 2 * H), lambda c, i: (c, 0, 0)),
        pl.BlockSpec((1, 2 * H, 2 * H), lambda c, i: (c, 0, 0)),
        pl.BlockSpec((1, 1, 2 * H), lambda c, i: (c, 0, 0)),
        pl.BlockSpec((1, 2 * H, 2 * H), lambda c, i: (c, 0, 0)),
        pl.BlockSpec((1, BR2, 2 * H), lambda c, i: (c, i, 0)),
    ],
    out_specs=pl.BlockSpec((1, BR2, 2 * H), lambda c, i: (c, i, 0)),
    out_shape=jax.ShapeDtypeStruct((2, PR, 2 * H), jnp.float32),
)


def _layer_body(acc_ref, y_ref, deg_ref, b_ref, w_ref, o_ref):
    dinv = lax.rsqrt(deg_ref[0] + 1.0)
    h = jnp.maximum(dinv * (acc_ref[0] + y_ref[0]) + b_ref[0], 0.0)
    o_ref[0] = jnp.dot(h, w_ref[0], preferred_element_type=jnp.float32) * dinv


_layer_call = pl.pallas_call(
    _layer_body,
    grid=(2, NPB2),
    in_specs=[
        pl.BlockSpec((1, BR2, 2 * H), lambda c, i: (c, i, 0)),
        pl.BlockSpec((1, BR2, 2 * H), lambda c, i: (c, i, 0)),
        pl.BlockSpec((1, BR2, 2 * H), lambda c, i: (c, i, 0)),
        pl.BlockSpec((1, 1, 2 * H), lambda c, i: (c, 0, 0)),
        pl.BlockSpec((1, 2 * H, 2 * H), lambda c, i: (c, 0, 0)),
    ],
    out_specs=pl.BlockSpec((1, BR2, 2 * H), lambda c, i: (c, i, 0)),
    out_shape=jax.ShapeDtypeStruct((2, PR, 2 * H), jnp.float32),
)


def _final_body(acc_ref, y_ref, deg_ref, b_ref, w1a_ref, w1b_ref, b1_ref,
                w2_ref, b2_ref, wh_ref, bh_ref, o_ref):
    dinv = lax.rsqrt(deg_ref[...] + 1.0)
    hf = jnp.maximum(dinv[0] * (acc_ref[0] + y_ref[0]) + b_ref[0], 0.0)
    hs = jnp.maximum(dinv[1] * (acc_ref[1] + y_ref[1]) + b_ref[1], 0.0)
    t = jnp.maximum(
        jnp.dot(hf, w1a_ref[...], preferred_element_type=jnp.float32)
        + jnp.dot(hs, w1b_ref[...], preferred_element_type=jnp.float32)
        + b1_ref[...], 0.0)
    u = jnp.dot(t, w2_ref[...], preferred_element_type=jnp.float32) + b2_ref[...]
    o_ref[...] = jnp.dot(u, wh_ref[...], preferred_element_type=jnp.float32) + bh_ref[...]


_final_call = pl.pallas_call(
    _final_body,
    grid=(NPB2,),
    in_specs=[
        pl.BlockSpec((2, BR2, 2 * H), lambda i: (0, i, 0)),
        pl.BlockSpec((2, BR2, 2 * H), lambda i: (0, i, 0)),
        pl.BlockSpec((2, BR2, 2 * H), lambda i: (0, i, 0)),
        pl.BlockSpec((2, 1, 2 * H), lambda i: (0, 0, 0)),
        pl.BlockSpec((2 * H, 2 * H), lambda i: (0, 0)),
        pl.BlockSpec((2 * H, 2 * H), lambda i: (0, 0)),
        pl.BlockSpec((1, 2 * H), lambda i: (0, 0)),
        pl.BlockSpec((2 * H, 2 * H), lambda i: (0, 0)),
        pl.BlockSpec((1, 2 * H), lambda i: (0, 0)),
        pl.BlockSpec((2 * H, 2 * H), lambda i: (0, 0)),
        pl.BlockSpec((1, 2 * H), lambda i: (0, 0)),
    ],
    out_specs=pl.BlockSpec((BR2, 2 * H), lambda i: (i, 0)),
    out_shape=jax.ShapeDtypeStruct((PR, 2 * H), jnp.float32),
)


def _pad_rows(x, rows):
    return jnp.concatenate(
        [x, jnp.zeros((rows - x.shape[0],) + x.shape[1:], x.dtype)], axis=0)


def _bd(w):
    z = jnp.zeros_like(w)
    return jnp.concatenate(
        [jnp.concatenate([w, z], 1), jnp.concatenate([z, w], 1)], 0)


def _tile2(b):
    return jnp.concatenate([b, b])


def kernel(front_x, front_edge_index, front_edge_attr, side_x, side_edge_index,
           side_edge_attr, f_enc_w1, f_enc_b1, f_enc_w2, f_enc_b2, f_conv_w0,
           f_conv_b0, f_conv_w1, f_conv_b1, f_conv_w2, f_conv_b2, s_enc_w1,
           s_enc_b1, s_enc_w2, s_enc_b2, s_conv_w0, s_conv_b0, s_conv_w1,
           s_conv_b1, s_conv_w2, s_conv_b2, fus_w1, fus_b1, fus_w2, fus_b2,
           no_w, no_b, nt_w, nt_b):
    f32 = jnp.float32

    pad = ERW * 128 - E
    sent = jnp.concatenate([jnp.zeros((1, pad), jnp.int32),
                            jnp.full((1, pad), N, jnp.int32)], axis=0)
    fe = jnp.concatenate([front_edge_index.astype(jnp.int32), sent],
                         axis=1).reshape(2, ERW, 128)
    se = jnp.concatenate([side_edge_index.astype(jnp.int32), sent],
                         axis=1).reshape(2, ERW, 128)

    ones128 = jnp.ones((128,), f32)
    zeros1 = jnp.zeros((NP,), f32)
    zerosH = jnp.zeros((NP, H), f32)

    degp = _deg_kernel(fe, se, ones128, zeros1).reshape(2, PR, 2 * H)

    xp = jnp.stack([_pad_rows(front_x, NP),
                    _pad_rows(side_x, NP)]).reshape(2, PR, 2 * D)
    ew1 = jnp.stack([_bd(f_enc_w1), _bd(s_enc_w1)])
    eb1 = jnp.stack([_tile2(f_enc_b1), _tile2(s_enc_b1)])[:, None, :]
    ew2 = jnp.stack([_bd(f_enc_w2), _bd(s_enc_w2)])
    eb2 = jnp.stack([_tile2(f_enc_b2), _tile2(s_enc_b2)])[:, None, :]
    cw = [jnp.stack([_bd(f_conv_w0), _bd(s_conv_w0)]),
          jnp.stack([_bd(f_conv_w1), _bd(s_conv_w1)]),
          jnp.stack([_bd(f_conv_w2), _bd(s_conv_w2)])]
    cb = [jnp.stack([_tile2(f_conv_b0), _tile2(s_conv_b0)])[:, None, :],
          jnp.stack([_tile2(f_conv_b1), _tile2(s_conv_b1)])[:, None, :],
          jnp.stack([_tile2(f_conv_b2), _tile2(s_conv_b2)])[:, None, :]]

    y = _enc_call(xp, ew1, eb1, ew2, eb2, cw[0], degp)
    accp = None
    for i in range(3):
        acc = _edge_kernel(y.reshape(2, NP, H), fe, se, zerosH)
        accp = acc.reshape(2, PR, 2 * H)
        if i < 2:
            y = _layer_call(accp, y, degp, cb[i], cw[i + 1])

    wh = jnp.zeros((H, H), f32).at[:, :32].set(no_w).at[:, 32:34].set(nt_w)
    bh = jnp.zeros((H,), f32).at[:32].set(no_b).at[32:34].set(nt_b)
    heads_p = _final_call(accp, y, degp, cb[2], _bd(fus_w1[:H]),
                          _bd(fus_w1[H:]), _tile2(fus_b1)[None, :],
                          _bd(fus_w2), _tile2(fus_b2)[None, :], _bd(wh),
                          _tile2(bh)[None, :])
    heads = heads_p.reshape(NP, H)
    return heads[:N, :32], heads[:N, 32:34]
